# baseline (speedup 1.0000x reference)
import functools

import jax
import jax.numpy as jnp
from jax import lax
from jax.experimental import pallas as pl
from jax.experimental.pallas import tpu as pltpu
from jax.experimental.pallas import tpu_sc as plsc

T = 8192
D = 768
E = 64
B = 256
NT = T // B
NI = NT + E - 1

DV = D // 2
DW = DV + 128
NWK = 32
TPW = T // NWK
CH = 64
CK = TPW // CH


def _lt(n, m, strict=True):
    r = lax.broadcasted_iota(jnp.int32, (n, m), 0)
    c = lax.broadcasted_iota(jnp.int32, (n, m), 1)
    return (r > c if strict else r >= c).astype(jnp.float32)


GB = 1024
NGB = T // GB


def _gate_body(x_ref, wg_ref, xb_ref, idx_ref):
    x = x_ref[...]
    u = lax.bitcast_convert_type(x, jnp.uint32)
    r = (u + jnp.uint32(0x7FFF) + ((u >> 16) & jnp.uint32(1))) >> 16
    packed = lax.bitcast_convert_type((r[:, DV:] << 16) | r[:, :DV], jnp.int32)
    wg = wg_ref[...]
    logits = jnp.dot(x, wg, preferred_element_type=jnp.float32)
    m = jnp.max(logits, axis=1, keepdims=True)
    iota_e = lax.broadcasted_iota(jnp.int32, (GB, E), 1)
    idx_ref[...] = jnp.min(jnp.where(logits >= m, iota_e, E), axis=1,
                           keepdims=True)
    g = 1.0 / jnp.sum(jnp.exp(logits - m), axis=1, keepdims=True)
    g128 = lax.bitcast_convert_type(g * jnp.ones((1, 128), jnp.float32),
                                    jnp.int32)
    xb_ref[...] = jnp.concatenate([packed, g128], axis=1)


def _gate(x, Wg):
    return pl.pallas_call(
        _gate_body,
        grid=(NGB,),
        in_specs=[
            pl.BlockSpec((GB, D), lambda i: (i, 0)),
            pl.BlockSpec((D, E), lambda i: (0, 0)),
        ],
        out_specs=(
            pl.BlockSpec((GB, DW), lambda i: (i, 0)),
            pl.BlockSpec((GB, 1), lambda i: (i, 0)),
        ),
        out_shape=(
            jax.ShapeDtypeStruct((T, DW), jnp.int32),
            jax.ShapeDtypeStruct((T, 1), jnp.int32),
        ),
    )(x, Wg)


def _route_body(idx_ref, pos_ref, it_ref, ie_ref, lo_ref, hi_ref):
    idx = idx_ref[...]
    PB = 256
    NB = T // PB
    iota_pb = lax.broadcasted_iota(jnp.int32, (PB, E), 1)
    ohs_list = [(iota_pb == idx[k * PB:(k + 1) * PB]).astype(jnp.float32)
                for k in range(NB)]
    bs = jnp.concatenate(
        [jnp.sum(o, axis=0, keepdims=True) for o in ohs_list], axis=0)
    bp = _lt(NB, NB) @ bs
    counts = jnp.sum(bs, axis=0, keepdims=True)
    offs = counts @ _lt(E, E).T
    lt_pb = _lt(PB, PB)
    pos_blocks = []
    for k in range(NB):
        oh_k = ohs_list[k]
        tot = lt_pb @ oh_k + bp[k:k + 1] + offs
        pos_blocks.append(jnp.sum(oh_k * tot, axis=1, keepdims=True))
    pos_ref[...] = jnp.concatenate(pos_blocks, axis=0).astype(jnp.int32)

    off_i = (counts @ _lt(E, E).T).astype(jnp.int32)
    cnt_i = counts.astype(jnp.int32)
    ft = off_i // B
    lt_ = (off_i + cnt_i - 1) // B
    items = jnp.where(cnt_i > 0, lt_ - ft + 1, 0)
    cum_x = (items.astype(jnp.float32) @ _lt(E, E).T).astype(jnp.int32)
    end_i = cum_x + items
    total = jnp.sum(items, axis=1, keepdims=True)

    s_col = lax.broadcasted_iota(jnp.int32, (NI, 1), 0)
    eid = jnp.sum((jnp.broadcast_to(end_i, (NI, E)) <=
                   jnp.broadcast_to(s_col, (NI, E))).astype(jnp.int32),
                  axis=1, keepdims=True)
    eid = jnp.minimum(eid, E - 1)
    ohs = (lax.broadcasted_iota(jnp.int32, (NI, E), 1) == eid).astype(jnp.float32)

    def sel(v):
        return jnp.sum(ohs * jnp.broadcast_to(v.astype(jnp.float32), (NI, E)),
                       axis=1, keepdims=True).astype(jnp.int32)

    valid = s_col < total[0, 0]
    tile = sel(ft) + (s_col - sel(cum_x))
    tile = jnp.where(valid, tile, NT - 1)
    s_off, s_cnt = sel(off_i), sel(cnt_i)
    lo = jnp.maximum(s_off, tile * B) - tile * B
    hi = jnp.minimum(s_off + s_cnt, (tile + 1) * B) - tile * B
    it_ref[...] = tile
    ie_ref[...] = eid
    lo_ref[...] = jnp.where(valid, lo, 0)
    hi_ref[...] = jnp.where(valid, hi, 0)


def _route(idx):
    return pl.pallas_call(
        _route_body,
        out_shape=(
            jax.ShapeDtypeStruct((T, 1), jnp.int32),
            jax.ShapeDtypeStruct((NI, 1), jnp.int32),
            jax.ShapeDtypeStruct((NI, 1), jnp.int32),
            jax.ShapeDtypeStruct((NI, 1), jnp.int32),
            jax.ShapeDtypeStruct((NI, 1), jnp.int32),
        ),
    )(idx)


def _experts_body(it_ref, ie_ref, lo_ref, hi_ref,
                  xs_ref, w1_ref, b1_ref, w2_ref, b2_ref, out_ref):
    s = pl.program_id(0)
    uw = lax.bitcast_convert_type(xs_ref[...], jnp.uint32)
    u = uw[:, :DV]
    gv = lax.bitcast_convert_type(uw[:, DV:], jnp.float32)
    x_lo = lax.bitcast_convert_type(u << 16, jnp.float32)
    x_hi = lax.bitcast_convert_type(u & jnp.uint32(0xFFFF0000), jnp.float32)
    xt = jnp.concatenate([x_lo, x_hi], axis=1).astype(jnp.bfloat16)
    w1 = w1_ref[0].astype(jnp.bfloat16)
    h = lax.dot_general(xt, w1, (((1,), (1,)), ((), ())),
                        preferred_element_type=jnp.float32) + b1_ref[0]
    h = jax.nn.gelu(h).astype(jnp.bfloat16)
    w2 = w2_ref[0].astype(jnp.bfloat16)
    y = lax.dot_general(h, w2, (((1,), (1,)), ((), ())),
                        preferred_element_type=jnp.float32) + b2_ref[0]
    y = y * gv[:, :1]
    rows = lax.broadcasted_iota(jnp.int32, (B, 1), 0)
    mask = (rows >= lo_ref[s, 0]) & (rows < hi_ref[s, 0])
    prev = it_ref[jnp.maximum(s - 1, 0), 0]
    first = jnp.logical_or(s == 0, it_ref[s, 0] != prev)

    @pl.when(first)
    def _():
        out_ref[...] = jnp.zeros_like(out_ref)

    out_ref[...] += jnp.where(mask, y, 0.0)


def _experts(xs, W1, b1, W2, b2, it, ie, lo, hi):
    grid_spec = pltpu.PrefetchScalarGridSpec(
        num_scalar_prefetch=4,
        grid=(NI,),
        in_specs=[
            pl.BlockSpec((B, DW), lambda s, it, ie, lo, hi: (it[s, 0], 0)),
            pl.BlockSpec((1, D, D), lambda s, it, ie, lo, hi: (ie[s, 0], 0, 0)),
            pl.BlockSpec((1, 1, D), lambda s, it, ie, lo, hi: (ie[s, 0], 0, 0)),
            pl.BlockSpec((1, D, D), lambda s, it, ie, lo, hi: (ie[s, 0], 0, 0)),
            pl.BlockSpec((1, 1, D), lambda s, it, ie, lo, hi: (ie[s, 0], 0, 0)),
        ],
        out_specs=pl.BlockSpec((B, D), lambda s, it, ie, lo, hi: (it[s, 0], 0)),
    )
    return pl.pallas_call(
        _experts_body,
        grid_spec=grid_spec,
        out_shape=jax.ShapeDtypeStruct((T, D), jnp.float32),
    )(it, ie, lo, hi, xs, W1, b1.reshape(E, 1, D), W2, b2.reshape(E, 1, D))


def _dispatch_body(x_hbm, pos_hbm, xs_hbm, idx_v, row_a, row_b,
                   lsa, lsb, ssa, ssb):
    wid = lax.axis_index("s") * 2 + lax.axis_index("c")
    base = wid * TPW
    pltpu.sync_copy(pos_hbm.at[wid], idx_v)
    rows = (row_a, row_b)
    lsem = (lsa, lsb)
    ssem = (ssa, ssb)
    ld = [None] * CK
    sc = [None] * CK
    for k in range(min(2, CK)):
        ld[k] = pltpu.async_copy(x_hbm.at[pl.ds(base + k * CH, CH)],
                                 rows[k % 2], lsem[k % 2])
    for k in range(CK):
        ld[k].wait()
        sc[k] = pltpu.async_copy(rows[k % 2], xs_hbm.at[idx_v.at[k]],
                                 ssem[k % 2])
        if k + 2 < CK:
            sc[k].wait()
            ld[k + 2] = pltpu.async_copy(
                x_hbm.at[pl.ds(base + (k + 2) * CH, CH)], rows[k % 2],
                lsem[k % 2])
    for k in range(max(0, CK - 2), CK):
        sc[k].wait()


def _combine_body(ys_hbm, pos_hbm, out_hbm, idx_v, row_a, row_b,
                  lsa, lsb, ssa, ssb):
    wid = lax.axis_index("s") * 2 + lax.axis_index("c")
    base = wid * TPW
    pltpu.sync_copy(pos_hbm.at[wid], idx_v)
    rows = (row_a, row_b)
    lsem = (lsa, lsb)
    ssem = (ssa, ssb)
    ld = [None] * CK
    st = [None] * CK
    for k in range(min(2, CK)):
        ld[k] = pltpu.async_copy(ys_hbm.at[idx_v.at[k]], rows[k % 2],
                                 lsem[k % 2])
    for k in range(CK):
        ld[k].wait()
        st[k] = pltpu.async_copy(rows[k % 2],
                                 out_hbm.at[pl.ds(base + k * CH, CH)],
                                 ssem[k % 2])
        if k + 2 < CK:
            st[k].wait()
            ld[k + 2] = pltpu.async_copy(ys_hbm.at[idx_v.at[k + 2]],
                                         rows[k % 2], lsem[k % 2])
    for k in range(max(0, CK - 2), CK):
        st[k].wait()


@functools.lru_cache(maxsize=None)
def _sc_kernels():
    mesh = plsc.VectorSubcoreMesh(core_axis_name="c", subcore_axis_name="s")
    dispatch = pl.kernel(
        _dispatch_body,
        out_type=jax.ShapeDtypeStruct((T, DW), jnp.int32),
        mesh=mesh,
        scratch_types=[
            pltpu.VMEM((CK, CH), jnp.int32),
            pltpu.VMEM((CH, DW), jnp.int32),
            pltpu.VMEM((CH, DW), jnp.int32),
        ] + [pltpu.SemaphoreType.DMA] * 4,
    )
    combine = pl.kernel(
        _combine_body,
        out_type=jax.ShapeDtypeStruct((T, D), jnp.float32),
        mesh=mesh,
        scratch_types=[
            pltpu.VMEM((CK, CH), jnp.int32),
            pltpu.VMEM((CH, D), jnp.float32),
            pltpu.VMEM((CH, D), jnp.float32),
        ] + [pltpu.SemaphoreType.DMA] * 4,
    )
    return dispatch, combine


def kernel(x, Wg, W1, b1, W2, b2):
    dispatch, combine = _sc_kernels()
    xb, idx = _gate(x, Wg)
    pos, it, ie, lo, hi = _route(idx)
    pos = pos.reshape(NWK, CK, CH)
    xs = dispatch(xb, pos)
    ys = _experts(xs, W1, b1, W2, b2, it, ie, lo, hi)
    return combine(ys, pos)


# speedup vs baseline: 1.1490x; 1.1490x over previous
import functools

import jax
import jax.numpy as jnp
from jax import lax
from jax.experimental import pallas as pl
from jax.experimental.pallas import tpu as pltpu
from jax.experimental.pallas import tpu_sc as plsc

T = 8192
D = 768
E = 64
B = 256
NT = T // B
NI = NT + E - 1

DV = D // 2
DW = DV + 128
NWK = 32
NBUF = 6
LA = 4
TPW = T // NWK
CH = 64
CK = TPW // CH


def _lt(n, m, strict=True):
    r = lax.broadcasted_iota(jnp.int32, (n, m), 0)
    c = lax.broadcasted_iota(jnp.int32, (n, m), 1)
    return (r > c if strict else r >= c).astype(jnp.float32)


GB = 1024
NGB = T // GB


def _gate_body(x_ref, wg_ref, xb_ref, idx_ref):
    x = x_ref[...]
    u = lax.bitcast_convert_type(x, jnp.uint32)
    r = (u + jnp.uint32(0x7FFF) + ((u >> 16) & jnp.uint32(1))) >> 16
    packed = lax.bitcast_convert_type((r[:, DV:] << 16) | r[:, :DV], jnp.int32)
    wg = wg_ref[...]
    logits = jnp.dot(x, wg, preferred_element_type=jnp.float32)
    m = jnp.max(logits, axis=1, keepdims=True)
    iota_e = lax.broadcasted_iota(jnp.int32, (GB, E), 1)
    idx_ref[...] = jnp.min(jnp.where(logits >= m, iota_e, E), axis=1,
                           keepdims=True)
    g = 1.0 / jnp.sum(jnp.exp(logits - m), axis=1, keepdims=True)
    g128 = lax.bitcast_convert_type(g * jnp.ones((1, 128), jnp.float32),
                                    jnp.int32)
    xb_ref[...] = jnp.concatenate([packed, g128], axis=1)


def _gate(x, Wg):
    return pl.pallas_call(
        _gate_body,
        grid=(NGB,),
        in_specs=[
            pl.BlockSpec((GB, D), lambda i: (i, 0)),
            pl.BlockSpec((D, E), lambda i: (0, 0)),
        ],
        out_specs=(
            pl.BlockSpec((GB, DW), lambda i: (i, 0)),
            pl.BlockSpec((GB, 1), lambda i: (i, 0)),
        ),
        out_shape=(
            jax.ShapeDtypeStruct((T, DW), jnp.int32),
            jax.ShapeDtypeStruct((T, 1), jnp.int32),
        ),
    )(x, Wg)


def _route_body(idx_ref, pos_ref, it_ref, ie_ref, lo_ref, hi_ref, ld_ref,
                sl_ref):
    idx = idx_ref[...]
    PB = 256
    NB = T // PB
    iota_pb = lax.broadcasted_iota(jnp.int32, (PB, E), 1)
    ohs_list = [(iota_pb == idx[k * PB:(k + 1) * PB]).astype(jnp.float32)
                for k in range(NB)]
    bs = jnp.concatenate(
        [jnp.sum(o, axis=0, keepdims=True) for o in ohs_list], axis=0)
    bp = _lt(NB, NB) @ bs
    counts = jnp.sum(bs, axis=0, keepdims=True)
    offs = counts @ _lt(E, E).T
    lt_pb = _lt(PB, PB)
    pos_blocks = []
    for k in range(NB):
        oh_k = ohs_list[k]
        tot = lt_pb @ oh_k + bp[k:k + 1] + offs
        pos_blocks.append(jnp.sum(oh_k * tot, axis=1, keepdims=True))
    pos_ref[...] = jnp.concatenate(pos_blocks, axis=0).astype(jnp.int32)

    off_i = (counts @ _lt(E, E).T).astype(jnp.int32)
    cnt_i = counts.astype(jnp.int32)
    ft = off_i // B
    lt_ = (off_i + cnt_i - 1) // B
    items = jnp.where(cnt_i > 0, lt_ - ft + 1, 0)
    cum_x = (items.astype(jnp.float32) @ _lt(E, E).T).astype(jnp.int32)
    end_i = cum_x + items
    total = jnp.sum(items, axis=1, keepdims=True)

    s_col = lax.broadcasted_iota(jnp.int32, (NI, 1), 0)
    eid = jnp.sum((jnp.broadcast_to(end_i, (NI, E)) <=
                   jnp.broadcast_to(s_col, (NI, E))).astype(jnp.int32),
                  axis=1, keepdims=True)
    eid = jnp.minimum(eid, E - 1)
    ohs = (lax.broadcasted_iota(jnp.int32, (NI, E), 1) == eid).astype(jnp.float32)

    def sel(v):
        return jnp.sum(ohs * jnp.broadcast_to(v.astype(jnp.float32), (NI, E)),
                       axis=1, keepdims=True).astype(jnp.int32)

    valid = s_col < total[0, 0]
    tile = sel(ft) + (s_col - sel(cum_x))
    tile = jnp.where(valid, tile, NT - 1)
    s_off, s_cnt = sel(off_i), sel(cnt_i)
    lo = jnp.maximum(s_off, tile * B) - tile * B
    hi = jnp.minimum(s_off + s_cnt, (tile + 1) * B) - tile * B
    it_ref[...] = tile
    ie_ref[...] = eid
    lo_ref[...] = jnp.where(valid, lo, 0)
    hi_ref[...] = jnp.where(valid, hi, 0)

    eid_prev = jnp.concatenate(
        [jnp.full((1, 1), -1, jnp.int32), eid[:NI - 1]], axis=0)
    ldv = (eid != eid_prev).astype(jnp.int32)
    rc = (_lt(NI, NI, strict=False) @ ldv.astype(jnp.float32)
          ).astype(jnp.int32) - 1
    ld_ref[...] = ldv
    sl_ref[...] = rc % NBUF


def _route(idx):
    return pl.pallas_call(
        _route_body,
        out_shape=(
            jax.ShapeDtypeStruct((T, 1), jnp.int32),
            jax.ShapeDtypeStruct((NI, 1), jnp.int32),
            jax.ShapeDtypeStruct((NI, 1), jnp.int32),
            jax.ShapeDtypeStruct((NI, 1), jnp.int32),
            jax.ShapeDtypeStruct((NI, 1), jnp.int32),
            jax.ShapeDtypeStruct((NI, 1), jnp.int32),
            jax.ShapeDtypeStruct((NI, 1), jnp.int32),
        ),
    )(idx)


def _experts_body(it_ref, ie_ref, lo_ref, hi_ref, ld_ref, sl_ref,
                  xs_ref, w1_hbm, b1_ref, w2_hbm, b2_ref, out_ref,
                  w1b, w2b, sem1, sem2):
    s = pl.program_id(0)

    def cps(item, slot):
        return (pltpu.make_async_copy(w1_hbm.at[ie_ref[item, 0]],
                                      w1b.at[slot], sem1.at[slot]),
                pltpu.make_async_copy(w2_hbm.at[ie_ref[item, 0]],
                                      w2b.at[slot], sem2.at[slot]))

    @pl.when(s == 0)
    def _():
        for j in range(LA):
            @pl.when(ld_ref[j, 0] == 1)
            def _():
                a, b = cps(j, sl_ref[j, 0])
                a.start()
                b.start()

    nxt = jnp.minimum(s + LA, NI - 1)

    @pl.when(jnp.logical_and(s + LA < NI, ld_ref[nxt, 0] == 1))
    def _():
        a, b = cps(nxt, sl_ref[nxt, 0])
        a.start()
        b.start()

    slot_s = sl_ref[s, 0]

    @pl.when(ld_ref[s, 0] == 1)
    def _():
        a, b = cps(s, slot_s)
        a.wait()
        b.wait()

    uw = lax.bitcast_convert_type(xs_ref[...], jnp.uint32)
    u = uw[:, :DV]
    gv = lax.bitcast_convert_type(uw[:, DV:], jnp.float32)
    x_lo = lax.bitcast_convert_type(u << 16, jnp.float32)
    x_hi = lax.bitcast_convert_type(u & jnp.uint32(0xFFFF0000), jnp.float32)
    xt = jnp.concatenate([x_lo, x_hi], axis=1).astype(jnp.bfloat16)
    w1 = w1b[slot_s].astype(jnp.bfloat16)
    h = lax.dot_general(xt, w1, (((1,), (1,)), ((), ())),
                        preferred_element_type=jnp.float32) + b1_ref[0]
    h = jax.nn.gelu(h).astype(jnp.bfloat16)
    w2 = w2b[slot_s].astype(jnp.bfloat16)
    y = lax.dot_general(h, w2, (((1,), (1,)), ((), ())),
                        preferred_element_type=jnp.float32) + b2_ref[0]
    y = y * gv[:, :1]
    rows = lax.broadcasted_iota(jnp.int32, (B, 1), 0)
    mask = (rows >= lo_ref[s, 0]) & (rows < hi_ref[s, 0])
    prev = it_ref[jnp.maximum(s - 1, 0), 0]
    first = jnp.logical_or(s == 0, it_ref[s, 0] != prev)

    @pl.when(first)
    def _():
        out_ref[...] = jnp.zeros_like(out_ref)

    out_ref[...] += jnp.where(mask, y, 0.0)


def _experts(xs, W1, b1, W2, b2, it, ie, lo, hi, ld, sl):
    grid_spec = pltpu.PrefetchScalarGridSpec(
        num_scalar_prefetch=6,
        grid=(NI,),
        in_specs=[
            pl.BlockSpec((B, DW),
                         lambda s, it, ie, lo, hi, ld, sl: (it[s, 0], 0)),
            pl.BlockSpec(memory_space=pltpu.HBM),
            pl.BlockSpec((1, 1, D),
                         lambda s, it, ie, lo, hi, ld, sl: (ie[s, 0], 0, 0)),
            pl.BlockSpec(memory_space=pltpu.HBM),
            pl.BlockSpec((1, 1, D),
                         lambda s, it, ie, lo, hi, ld, sl: (ie[s, 0], 0, 0)),
        ],
        out_specs=pl.BlockSpec((B, D),
                               lambda s, it, ie, lo, hi, ld, sl: (it[s, 0], 0)),
        scratch_shapes=[
            pltpu.VMEM((NBUF, D, D), jnp.float32),
            pltpu.VMEM((NBUF, D, D), jnp.float32),
            pltpu.SemaphoreType.DMA((NBUF,)),
            pltpu.SemaphoreType.DMA((NBUF,)),
        ],
    )
    return pl.pallas_call(
        _experts_body,
        grid_spec=grid_spec,
        out_shape=jax.ShapeDtypeStruct((T, D), jnp.float32),
    )(it, ie, lo, hi, ld, sl, xs, W1, b1.reshape(E, 1, D), W2,
      b2.reshape(E, 1, D))


def _dispatch_body(x_hbm, pos_hbm, xs_hbm, idx_v, row_a, row_b,
                   lsa, lsb, ssa, ssb):
    wid = lax.axis_index("s") * 2 + lax.axis_index("c")
    base = wid * TPW
    pltpu.sync_copy(pos_hbm.at[wid], idx_v)
    rows = (row_a, row_b)
    lsem = (lsa, lsb)
    ssem = (ssa, ssb)
    ld = [None] * CK
    sc = [None] * CK
    for k in range(min(2, CK)):
        ld[k] = pltpu.async_copy(x_hbm.at[pl.ds(base + k * CH, CH)],
                                 rows[k % 2], lsem[k % 2])
    for k in range(CK):
        ld[k].wait()
        sc[k] = pltpu.async_copy(rows[k % 2], xs_hbm.at[idx_v.at[k]],
                                 ssem[k % 2])
        if k + 2 < CK:
            sc[k].wait()
            ld[k + 2] = pltpu.async_copy(
                x_hbm.at[pl.ds(base + (k + 2) * CH, CH)], rows[k % 2],
                lsem[k % 2])
    for k in range(max(0, CK - 2), CK):
        sc[k].wait()


def _combine_body(ys_hbm, pos_hbm, out_hbm, idx_v, row_a, row_b,
                  lsa, lsb, ssa, ssb):
    wid = lax.axis_index("s") * 2 + lax.axis_index("c")
    base = wid * TPW
    pltpu.sync_copy(pos_hbm.at[wid], idx_v)
    rows = (row_a, row_b)
    lsem = (lsa, lsb)
    ssem = (ssa, ssb)
    ld = [None] * CK
    st = [None] * CK
    for k in range(min(2, CK)):
        ld[k] = pltpu.async_copy(ys_hbm.at[idx_v.at[k]], rows[k % 2],
                                 lsem[k % 2])
    for k in range(CK):
        ld[k].wait()
        st[k] = pltpu.async_copy(rows[k % 2],
                                 out_hbm.at[pl.ds(base + k * CH, CH)],
                                 ssem[k % 2])
        if k + 2 < CK:
            st[k].wait()
            ld[k + 2] = pltpu.async_copy(ys_hbm.at[idx_v.at[k + 2]],
                                         rows[k % 2], lsem[k % 2])
    for k in range(max(0, CK - 2), CK):
        st[k].wait()


@functools.lru_cache(maxsize=None)
def _sc_kernels():
    mesh = plsc.VectorSubcoreMesh(core_axis_name="c", subcore_axis_name="s")
    dispatch = pl.kernel(
        _dispatch_body,
        out_type=jax.ShapeDtypeStruct((T, DW), jnp.int32),
        mesh=mesh,
        scratch_types=[
            pltpu.VMEM((CK, CH), jnp.int32),
            pltpu.VMEM((CH, DW), jnp.int32),
            pltpu.VMEM((CH, DW), jnp.int32),
        ] + [pltpu.SemaphoreType.DMA] * 4,
    )
    combine = pl.kernel(
        _combine_body,
        out_type=jax.ShapeDtypeStruct((T, D), jnp.float32),
        mesh=mesh,
        scratch_types=[
            pltpu.VMEM((CK, CH), jnp.int32),
            pltpu.VMEM((CH, D), jnp.float32),
            pltpu.VMEM((CH, D), jnp.float32),
        ] + [pltpu.SemaphoreType.DMA] * 4,
    )
    return dispatch, combine


def kernel(x, Wg, W1, b1, W2, b2):
    dispatch, combine = _sc_kernels()
    xb, idx = _gate(x, Wg)
    pos, it, ie, lo, hi, ld, sl = _route(idx)
    pos = pos.reshape(NWK, CK, CH)
    xs = dispatch(xb, pos)
    ys = _experts(xs, W1, b1, W2, b2, it, ie, lo, hi, ld, sl)
    return combine(ys, pos)
